# baseline (speedup 1.0000x reference)
import jax
import jax.numpy as jnp
from jax.experimental import pallas as pl
from jax.experimental.pallas import tpu as pltpu

N = 2048
K = 128
BM = 512
BN = 512
BK = 512


def _grad_mask_kernel(w_ik, a_jk, a_ki, w_kj, out_ref):
    i = pl.program_id(0)
    j = pl.program_id(1)
    k = pl.program_id(2)

    @pl.when(k == 0)
    def _zero():
        out_ref[...] = jnp.zeros_like(out_ref)

    d1 = jax.lax.dot_general(
        w_ik[...], a_jk[...], (((1,), (1,)), ((), ())),
        preferred_element_type=jnp.float32)
    d2 = jax.lax.dot_general(
        a_ki[...], w_kj[...], (((0,), (0,)), ((), ())),
        preferred_element_type=jnp.float32)
    out_ref[...] += d1 + d2

    @pl.when(k == pl.num_programs(2) - 1)
    def _mask():
        rows = i * BM + jax.lax.broadcasted_iota(jnp.int32, (BM, BN), 0)
        cols = j * BN + jax.lax.broadcasted_iota(jnp.int32, (BM, BN), 1)
        out_ref[...] = jnp.where(cols < rows, out_ref[...], jnp.inf)


def _masked_gradient(adj, W):
    grid = (N // BM, N // BN, N // BK)
    return pl.pallas_call(
        _grad_mask_kernel,
        grid=grid,
        in_specs=[
            pl.BlockSpec((BM, BK), lambda i, j, k: (i, k)),
            pl.BlockSpec((BN, BK), lambda i, j, k: (j, k)),
            pl.BlockSpec((BK, BM), lambda i, j, k: (k, i)),
            pl.BlockSpec((BK, BN), lambda i, j, k: (k, j)),
        ],
        out_specs=pl.BlockSpec((BM, BN), lambda i, j, k: (i, j)),
        out_shape=jax.ShapeDtypeStruct((N, N), jnp.float32),
    )(W, adj, adj, W)


def kernel(adj, W):
    g = _masked_gradient(adj, W)
    flat = g.reshape(-1)
    neg_vals, idx = jax.lax.top_k(-flat, K)
    actions = jnp.stack([idx // N, idx % N], axis=-1)
    return (actions, jnp.zeros((1,), dtype=jnp.float32))


# speedup vs baseline: 10.8343x; 10.8343x over previous
import functools

import numpy as np

import jax
import jax.numpy as jnp
from jax import lax
from jax.experimental import pallas as pl
from jax.experimental.pallas import tpu as pltpu
from jax.experimental.pallas import tpu_sc as plsc

N = 2048
K = 128
BM = 512
BN = 512
BK = 512

NW = 32
SHARD = (N * N) // NW
CHUNK = 16384
NCHUNK = SHARD // CHUNK
NBUCKET = 2048
CAP_T = 512
CAND = NW * CAP_T
IMAX = np.int32(0x7FFFFFFF)


def _grad_mask_kernel(w_ik, a_jk, a_ki, w_kj, out_ref, acc_ref):
    i = pl.program_id(0)
    j = pl.program_id(1)
    k = pl.program_id(2)

    @pl.when(k == 0)
    def _zero():
        acc_ref[...] = jnp.zeros_like(acc_ref)

    d1 = lax.dot_general(
        w_ik[...], a_jk[...], (((1,), (1,)), ((), ())),
        preferred_element_type=jnp.float32)
    d2 = lax.dot_general(
        a_ki[...], w_kj[...], (((0,), (0,)), ((), ())),
        preferred_element_type=jnp.float32)
    acc_ref[...] += d1 + d2

    @pl.when(k == pl.num_programs(2) - 1)
    def _mask():
        rows = i * BM + lax.broadcasted_iota(jnp.int32, (BM, BN), 0)
        cols = j * BN + lax.broadcasted_iota(jnp.int32, (BM, BN), 1)
        g = jnp.where(cols < rows, acc_ref[...], jnp.inf)
        u = lax.bitcast_convert_type(g, jnp.int32)
        skey = u ^ ((u >> 31) & IMAX)
        out_ref[...] = skey


def _masked_gradient_keys(adj, W):
    grid = (N // BM, N // BN, N // BK)
    return pl.pallas_call(
        _grad_mask_kernel,
        grid=grid,
        in_specs=[
            pl.BlockSpec((BM, BK), lambda i, j, k: (i, k)),
            pl.BlockSpec((BN, BK), lambda i, j, k: (j, k)),
            pl.BlockSpec((BK, BM), lambda i, j, k: (k, i)),
            pl.BlockSpec((BK, BN), lambda i, j, k: (k, j)),
        ],
        out_specs=pl.BlockSpec((BM, BN), lambda i, j, k: (i, j)),
        out_shape=jax.ShapeDtypeStruct((N, N), jnp.int32),
        scratch_shapes=[pltpu.VMEM((BM, BN), jnp.float32)],
    )(W, adj, adj, W)


def _worker_id():
    return lax.axis_index("s") * 2 + lax.axis_index("c")


def _histogram_sc(kflat):
    mesh = plsc.VectorSubcoreMesh(core_axis_name="c", subcore_axis_name="s")

    @functools.partial(
        pl.kernel,
        mesh=mesh,
        compiler_params=pltpu.CompilerParams(needs_layout_passes=False),
        out_type=jax.ShapeDtypeStruct((NW * NBUCKET,), jnp.int32),
        scratch_types=[
            pltpu.VMEM((CHUNK,), jnp.int32),
            pltpu.VMEM((16 * NBUCKET,), jnp.int32),
            pltpu.VMEM((NBUCKET,), jnp.int32),
        ],
    )
    def hist_kernel(k_hbm, out_hbm, chunk_v, hist16, hist):
        wid = _worker_id()
        base = wid * SHARD
        lanes = lax.iota(jnp.int32, 16)
        zero16 = jnp.zeros((16,), jnp.int32)
        ones16 = jnp.ones((16,), jnp.int32)

        def z_body(b, _):
            hist16[pl.ds(b * 16, 16)] = zero16
            return 0
        lax.fori_loop(0, (16 * NBUCKET) // 16, z_body, 0)

        def chunk_body(c, _):
            pltpu.sync_copy(k_hbm.at[pl.ds(base + c * CHUNK, CHUNK)], chunk_v)

            def v_body(i, _):
                sk = chunk_v[pl.ds(i * 16, 16)]
                b = (sk >> 21) + 1024
                slot = lanes * NBUCKET + b
                plsc.addupdate_scatter(hist16, [slot], ones16)
                return 0
            lax.fori_loop(0, CHUNK // 16, v_body, 0)
            return 0
        lax.fori_loop(0, NCHUNK, chunk_body, 0)

        def merge_body(b, _):
            acc = zero16
            for l in range(16):
                acc = acc + hist16[pl.ds(l * NBUCKET + b * 16, 16)]
            hist[pl.ds(b * 16, 16)] = acc
            return 0
        lax.fori_loop(0, NBUCKET // 16, merge_body, 0)

        pltpu.sync_copy(hist, out_hbm.at[pl.ds(wid * NBUCKET, NBUCKET)])

    return hist_kernel(kflat)


def _extract_sc(kflat, hists):
    mesh = plsc.VectorSubcoreMesh(core_axis_name="c", subcore_axis_name="s")

    @functools.partial(
        pl.kernel,
        mesh=mesh,
        compiler_params=pltpu.CompilerParams(needs_layout_passes=False),
        out_type=(jax.ShapeDtypeStruct((CAND,), jnp.int32),
                  jax.ShapeDtypeStruct((CAND,), jnp.int32)),
        scratch_types=[
            pltpu.VMEM((CHUNK,), jnp.int32),
            pltpu.VMEM((NW * NBUCKET,), jnp.int32),
            pltpu.VMEM((NBUCKET,), jnp.int32),
            pltpu.VMEM((CAP_T + 16,), jnp.int32),
            pltpu.VMEM((CAP_T + 16,), jnp.int32),
        ],
    )
    def ext_kernel(k_hbm, h_hbm, out_v, out_i, chunk_v, allh, hist, cv, ci):
        wid = _worker_id()
        base = wid * SHARD
        lanes = lax.iota(jnp.int32, 16)
        zero16 = jnp.zeros((16,), jnp.int32)
        imax16 = jnp.full((16,), IMAX, jnp.int32)

        pltpu.sync_copy(h_hbm, allh)

        def gmerge(b, _):
            acc = zero16
            for w in range(NW):
                acc = acc + allh[pl.ds(w * NBUCKET + b * 16, 16)]
            hist[pl.ds(b * 16, 16)] = acc
            return 0
        lax.fori_loop(0, NBUCKET // 16, gmerge, 0)

        def scan_body(b, carry):
            cum, bstar = carry
            hv = hist[pl.ds(b * 16, 16)]
            cums = plsc.cumsum(hv) + cum
            idxs = b * 16 + lanes
            cand = jnp.where(cums >= K, idxs, jnp.int32(NBUCKET))
            return (cum + jnp.sum(hv), jnp.minimum(bstar, jnp.min(cand)))
        _, bstar = lax.fori_loop(
            0, NBUCKET // 16, scan_body,
            (jnp.int32(0), jnp.int32(NBUCKET)))
        key_hi = (bstar - 1023) << 21

        def c_init(b, _):
            cv[pl.ds(b * 16, 16)] = imax16
            ci[pl.ds(b * 16, 16)] = zero16
            return 0
        lax.fori_loop(0, (CAP_T + 16) // 16, c_init, 0)

        def chunk_body(c, ptr):
            pltpu.sync_copy(k_hbm.at[pl.ds(base + c * CHUNK, CHUNK)], chunk_v)

            def v_body(i, ptr):
                sk = chunk_v[pl.ds(i * 16, 16)]
                m = sk < key_hi
                idxv = (base + c * CHUNK + i * 16) + lanes
                p = jnp.minimum(ptr, CAP_T)
                plsc.store_compressed(cv.at[pl.ds(p, 16)], sk, mask=m)
                plsc.store_compressed(ci.at[pl.ds(p, 16)], idxv, mask=m)
                cnt16 = plsc.all_reduce_population_count(m)
                return ptr + cnt16[0]
            return lax.fori_loop(0, CHUNK // 16, v_body, ptr)
        lax.fori_loop(0, NCHUNK, chunk_body, jnp.int32(0))

        pltpu.sync_copy(cv.at[pl.ds(0, CAP_T)],
                        out_v.at[pl.ds(wid * CAP_T, CAP_T)])
        pltpu.sync_copy(ci.at[pl.ds(0, CAP_T)],
                        out_i.at[pl.ds(wid * CAP_T, CAP_T)])

    return ext_kernel(kflat, hists)


def _final_kernel(v_ref, i_ref, row_ref, col_ref):
    v = v_ref[...]
    ii = i_ref[...]
    lane = lax.broadcasted_iota(jnp.int32, (1, K), 1)

    def body(t, carry):
        v, rows, cols = carry
        m = jnp.min(v)
        am = jnp.min(jnp.where(v == m, ii, IMAX))
        pred = lane == t
        rows = jnp.where(pred, am >> 11, rows)
        cols = jnp.where(pred, am & (N - 1), cols)
        v = jnp.where(ii == am, IMAX, v)
        return (v, rows, cols)

    zero = jnp.zeros((1, K), jnp.int32)
    _, rows, cols = lax.fori_loop(0, K, body, (v, zero, zero))
    row_ref[...] = rows
    col_ref[...] = cols


def _final_topk(cv, ci):
    return pl.pallas_call(
        _final_kernel,
        out_shape=(jax.ShapeDtypeStruct((1, K), jnp.int32),
                   jax.ShapeDtypeStruct((1, K), jnp.int32)),
    )(cv.reshape(K, CAND // K), ci.reshape(K, CAND // K))


def kernel(adj, W):
    keys = _masked_gradient_keys(adj, W)
    kflat = keys.reshape(-1)
    hists = _histogram_sc(kflat)
    cv, ci = _extract_sc(kflat, hists)
    rows, cols = _final_topk(cv, ci)
    actions = jnp.stack([rows[0], cols[0]], axis=-1)
    return (actions, jnp.zeros((1,), dtype=jnp.float32))


# speedup vs baseline: 22.0725x; 2.0373x over previous
import functools

import numpy as np

import jax
import jax.numpy as jnp
from jax import lax
from jax.experimental import pallas as pl
from jax.experimental.pallas import tpu as pltpu
from jax.experimental.pallas import tpu_sc as plsc

N = 2048
K = 128
BM = 512
BN = 512
BK = 512

NW = 32
ROWS_W = N // NW
CAP_T = 256
CAND = NW * CAP_T
IMAX = np.int32(0x7FFFFFFF)
IMIN = np.int32(-0x80000000)
KEY_INF = np.int32(0x7F800000)


def _grad_mask_kernel(w_ik, a_jk, a_ki, w_kj, out_ref, rmin_ref, acc_ref):
    i = pl.program_id(0)
    j = pl.program_id(1)
    k = pl.program_id(2)
    k_last = pl.num_programs(2) - 1

    @pl.when(j <= i)
    def _compute():
        @pl.when(k == 0)
        def _zero():
            acc_ref[...] = jnp.zeros_like(acc_ref)

        d1 = lax.dot_general(
            w_ik[...], a_jk[...], (((1,), (1,)), ((), ())),
            preferred_element_type=jnp.float32)
        d2 = lax.dot_general(
            a_ki[...], w_kj[...], (((0,), (0,)), ((), ())),
            preferred_element_type=jnp.float32)
        acc_ref[...] += d1 + d2

        @pl.when(k == k_last)
        def _mask():
            rows = i * BM + lax.broadcasted_iota(jnp.int32, (BM, BN), 0)
            cols = j * BN + lax.broadcasted_iota(jnp.int32, (BM, BN), 1)
            g = jnp.where(cols < rows, acc_ref[...], jnp.inf)
            u = lax.bitcast_convert_type(g, jnp.int32)
            skey = u ^ ((u >> 31) & IMAX)
            out_ref[...] = skey
            bmin = jnp.min(skey, axis=1, keepdims=True)

            @pl.when(j == 0)
            def _init():
                rmin_ref[pl.ds(i * BM, BM), :] = bmin

            @pl.when(j > 0)
            def _acc():
                rmin_ref[pl.ds(i * BM, BM), :] = jnp.minimum(
                    rmin_ref[pl.ds(i * BM, BM), :], bmin)

    @pl.when((j > i) & (k == k_last))
    def _upper():
        out_ref[...] = jnp.full((BM, BN), KEY_INF, jnp.int32)


def _masked_gradient_keys(adj, W):
    grid = (N // BM, N // BN, N // BK)
    return pl.pallas_call(
        _grad_mask_kernel,
        grid=grid,
        in_specs=[
            pl.BlockSpec((BM, BK), lambda i, j, k: (i, k)),
            pl.BlockSpec((BN, BK), lambda i, j, k: (j, k)),
            pl.BlockSpec((BK, BM), lambda i, j, k: (k, i)),
            pl.BlockSpec((BK, BN), lambda i, j, k: (k, j)),
        ],
        out_specs=(
            pl.BlockSpec((BM, BN), lambda i, j, k: (i, j)),
            pl.BlockSpec((N, 1), lambda i, j, k: (0, 0)),
        ),
        out_shape=(jax.ShapeDtypeStruct((N, N), jnp.int32),
                   jax.ShapeDtypeStruct((N, 1), jnp.int32)),
        scratch_shapes=[pltpu.VMEM((BM, BN), jnp.float32)],
    )(W, adj, adj, W)


def _extract_sc(kflat, rmin):
    mesh = plsc.VectorSubcoreMesh(core_axis_name="c", subcore_axis_name="s")

    @functools.partial(
        pl.kernel,
        mesh=mesh,
        compiler_params=pltpu.CompilerParams(needs_layout_passes=False),
        out_type=(jax.ShapeDtypeStruct((CAND,), jnp.int32),
                  jax.ShapeDtypeStruct((CAND,), jnp.int32)),
        scratch_types=[
            pltpu.VMEM((N,), jnp.int32),
            pltpu.VMEM((N,), jnp.int32),
            pltpu.VMEM((CAP_T + 16,), jnp.int32),
            pltpu.VMEM((CAP_T + 16,), jnp.int32),
        ],
    )
    def ext_kernel(k_hbm, rm_hbm, out_v, out_i, rmv, rowbuf, cv, ci):
        wid = _worker_id()
        lanes = lax.iota(jnp.int32, 16)
        zero16 = jnp.zeros((16,), jnp.int32)
        imax16 = jnp.full((16,), IMAX, jnp.int32)

        pltpu.sync_copy(rm_hbm, rmv)

        def bs_body(_, carry):
            lo, hi = carry
            mid = lo + lax.shift_right_logical(hi - lo, 1)

            def cnt(b, acc):
                rv = rmv[pl.ds(b * 16, 16)]
                return acc + jnp.where(rv <= mid, 1, 0).astype(jnp.int32)
            acc = lax.fori_loop(0, N // 16, cnt, zero16, unroll=8)
            big = jnp.sum(acc) >= K
            return (jnp.where(big, lo, mid), jnp.where(big, mid, hi))
        _, thresh = lax.fori_loop(0, 32, bs_body, (IMIN, IMAX))

        def c_init(b, _):
            cv[pl.ds(b * 16, 16)] = imax16
            ci[pl.ds(b * 16, 16)] = zero16
            return 0
        lax.fori_loop(0, (CAP_T + 16) // 16, c_init, 0)

        def scan_row(row, ptr):
            pltpu.sync_copy(k_hbm.at[pl.ds(row * N, N)], rowbuf)

            def v_body(i, ptr):
                sk = rowbuf[pl.ds(i * 16, 16)]
                m = sk <= thresh
                idxv = (row * N + i * 16) + lanes
                p = jnp.minimum(ptr, CAP_T)
                plsc.store_compressed(cv.at[pl.ds(p, 16)], sk, mask=m)
                plsc.store_compressed(ci.at[pl.ds(p, 16)], idxv, mask=m)
                cnt16 = plsc.all_reduce_population_count(m)
                return ptr + cnt16[0]
            return lax.fori_loop(0, N // 16, v_body, ptr)

        def group_body(g, ptr):
            rv = rmv[pl.ds(wid * ROWS_W + g * 16, 16)]
            for l in range(16):
                row = wid * ROWS_W + g * 16 + l
                ptr = lax.cond(rv[l] <= thresh,
                               functools.partial(scan_row, row),
                               lambda p: p, ptr)
            return ptr
        lax.fori_loop(0, ROWS_W // 16, group_body, jnp.int32(0))

        pltpu.sync_copy(cv.at[pl.ds(0, CAP_T)],
                        out_v.at[pl.ds(wid * CAP_T, CAP_T)])
        pltpu.sync_copy(ci.at[pl.ds(0, CAP_T)],
                        out_i.at[pl.ds(wid * CAP_T, CAP_T)])

    return ext_kernel(kflat, rmin)


def _worker_id():
    return lax.axis_index("s") * 2 + lax.axis_index("c")


def _final_kernel(v_ref, i_ref, row_ref, col_ref):
    v = v_ref[...]
    ii = i_ref[...]
    lane = lax.broadcasted_iota(jnp.int32, (1, K), 1)

    def body(t, carry):
        v, rows, cols = carry
        m = jnp.min(v)
        am = jnp.min(jnp.where(v == m, ii, IMAX))
        pred = lane == t
        rows = jnp.where(pred, am >> 11, rows)
        cols = jnp.where(pred, am & (N - 1), cols)
        v = jnp.where(ii == am, IMAX, v)
        return (v, rows, cols)

    zero = jnp.zeros((1, K), jnp.int32)
    _, rows, cols = lax.fori_loop(0, K, body, (v, zero, zero))
    row_ref[...] = rows
    col_ref[...] = cols


def _final_topk(cv, ci):
    return pl.pallas_call(
        _final_kernel,
        out_shape=(jax.ShapeDtypeStruct((1, K), jnp.int32),
                   jax.ShapeDtypeStruct((1, K), jnp.int32)),
    )(cv.reshape(CAND // K, K), ci.reshape(CAND // K, K))


def kernel(adj, W):
    keys, rmin = _masked_gradient_keys(adj, W)
    cv, ci = _extract_sc(keys.reshape(-1), rmin.reshape(-1))
    rows, cols = _final_topk(cv, ci)
    actions = jnp.stack([rows[0], cols[0]], axis=-1)
    return (actions, jnp.zeros((1,), dtype=jnp.float32))


# speedup vs baseline: 30.8388x; 1.3972x over previous
import functools

import numpy as np

import jax
import jax.numpy as jnp
from jax import lax
from jax.experimental import pallas as pl
from jax.experimental.pallas import tpu as pltpu
from jax.experimental.pallas import tpu_sc as plsc

N = 2048
K = 128
BM = 1024
BN = 1024
BK = 512

NW = 32
ROWS_W = N // NW
CAP_T = 256
CAND = NW * CAP_T
IMAX = np.int32(0x7FFFFFFF)
IMIN = np.int32(-0x80000000)
KEY_INF = np.int32(0x7F800000)


def _grad_mask_kernel(w_ik, a_jk, a_ki, w_kj, out_ref, rmin_ref, acc_ref):
    i = pl.program_id(0)
    j = pl.program_id(1)
    k = pl.program_id(2)
    k_last = pl.num_programs(2) - 1

    @pl.when(j <= i)
    def _compute():
        @pl.when(k == 0)
        def _zero():
            acc_ref[...] = jnp.zeros_like(acc_ref)

        d1 = lax.dot_general(
            w_ik[...], a_jk[...], (((1,), (1,)), ((), ())),
            preferred_element_type=jnp.float32)
        d2 = lax.dot_general(
            a_ki[...], w_kj[...], (((0,), (0,)), ((), ())),
            preferred_element_type=jnp.float32)
        acc_ref[...] += d1 + d2

        @pl.when(k == k_last)
        def _mask():
            rows = i * BM + lax.broadcasted_iota(jnp.int32, (BM, BN), 0)
            cols = j * BN + lax.broadcasted_iota(jnp.int32, (BM, BN), 1)
            g = jnp.where(cols < rows, acc_ref[...], jnp.inf)
            u = lax.bitcast_convert_type(g, jnp.int32)
            skey = u ^ ((u >> 31) & IMAX)
            out_ref[...] = skey
            bmin = jnp.min(skey, axis=1, keepdims=True)

            @pl.when(j == 0)
            def _init():
                rmin_ref[pl.ds(i * BM, BM), :] = bmin

            @pl.when(j > 0)
            def _acc():
                rmin_ref[pl.ds(i * BM, BM), :] = jnp.minimum(
                    rmin_ref[pl.ds(i * BM, BM), :], bmin)

    @pl.when((j > i) & (k == k_last))
    def _upper():
        out_ref[...] = jnp.full((BM, BN), KEY_INF, jnp.int32)


def _masked_gradient_keys(adj, W):
    grid = (N // BM, N // BN, N // BK)
    return pl.pallas_call(
        _grad_mask_kernel,
        grid=grid,
        in_specs=[
            pl.BlockSpec((BM, BK), lambda i, j, k: (i, k)),
            pl.BlockSpec((BN, BK), lambda i, j, k: (j, k)),
            pl.BlockSpec((BK, BM), lambda i, j, k: (k, i)),
            pl.BlockSpec((BK, BN), lambda i, j, k: (k, j)),
        ],
        out_specs=(
            pl.BlockSpec((BM, BN), lambda i, j, k: (i, j)),
            pl.BlockSpec((N, 1), lambda i, j, k: (0, 0)),
        ),
        out_shape=(jax.ShapeDtypeStruct((N, N), jnp.int32),
                   jax.ShapeDtypeStruct((N, 1), jnp.int32)),
        scratch_shapes=[pltpu.VMEM((BM, BN), jnp.float32)],
    )(W, adj, adj, W)


def _extract_sc(kflat, rmin):
    mesh = plsc.VectorSubcoreMesh(core_axis_name="c", subcore_axis_name="s")

    @functools.partial(
        pl.kernel,
        mesh=mesh,
        compiler_params=pltpu.CompilerParams(needs_layout_passes=False),
        out_type=(jax.ShapeDtypeStruct((CAND,), jnp.int32),
                  jax.ShapeDtypeStruct((CAND,), jnp.int32)),
        scratch_types=[
            pltpu.VMEM((N,), jnp.int32),
            pltpu.VMEM((N,), jnp.int32),
            pltpu.VMEM((CAP_T + 16,), jnp.int32),
            pltpu.VMEM((CAP_T + 16,), jnp.int32),
        ],
    )
    def ext_kernel(k_hbm, rm_hbm, out_v, out_i, rmv, rowbuf, cv, ci):
        wid = _worker_id()
        lanes = lax.iota(jnp.int32, 16)
        zero16 = jnp.zeros((16,), jnp.int32)
        imax16 = jnp.full((16,), IMAX, jnp.int32)

        pltpu.sync_copy(rm_hbm, rmv)

        def bs_body(_, carry):
            lo, hi = carry
            mid = lo + lax.shift_right_logical(hi - lo, 1)

            def cnt(b, acc):
                rv = rmv[pl.ds(b * 16, 16)]
                return acc + jnp.where(rv <= mid, 1, 0).astype(jnp.int32)
            acc = lax.fori_loop(0, N // 16, cnt, zero16, unroll=8)
            big = jnp.sum(acc) >= K
            return (jnp.where(big, lo, mid), jnp.where(big, mid, hi))
        _, thresh = lax.fori_loop(0, 32, bs_body, (IMIN, IMAX))

        def c_init(b, _):
            cv[pl.ds(b * 16, 16)] = imax16
            ci[pl.ds(b * 16, 16)] = zero16
            return 0
        lax.fori_loop(0, (CAP_T + 16) // 16, c_init, 0)

        def scan_row(row, ptr):
            pltpu.sync_copy(k_hbm.at[row], rowbuf)

            def v_body(i, ptr):
                sk = rowbuf[pl.ds(i * 16, 16)]
                m = sk <= thresh
                idxv = (row * N + i * 16) + lanes
                p = jnp.minimum(ptr, CAP_T)
                plsc.store_compressed(cv.at[pl.ds(p, 16)], sk, mask=m)
                plsc.store_compressed(ci.at[pl.ds(p, 16)], idxv, mask=m)
                cnt16 = plsc.all_reduce_population_count(m)
                return ptr + cnt16[0]
            return lax.fori_loop(0, N // 16, v_body, ptr)

        def group_body(g, ptr):
            rv = rmv[pl.ds(wid * ROWS_W + g * 16, 16)]
            for l in range(16):
                row = wid * ROWS_W + g * 16 + l
                ptr = lax.cond(rv[l] <= thresh,
                               functools.partial(scan_row, row),
                               lambda p: p, ptr)
            return ptr
        lax.fori_loop(0, ROWS_W // 16, group_body, jnp.int32(0))

        pltpu.sync_copy(cv.at[pl.ds(0, CAP_T)],
                        out_v.at[pl.ds(wid * CAP_T, CAP_T)])
        pltpu.sync_copy(ci.at[pl.ds(0, CAP_T)],
                        out_i.at[pl.ds(wid * CAP_T, CAP_T)])

    return ext_kernel(kflat, rmin)


def _worker_id():
    return lax.axis_index("s") * 2 + lax.axis_index("c")


def _final_kernel(v_ref, i_ref, row_ref, col_ref):
    v = v_ref[...]
    ii = i_ref[...]
    lane = lax.broadcasted_iota(jnp.int32, (1, K), 1)

    def body(t, carry):
        v, rows, cols = carry
        m = jnp.min(v)
        am = jnp.min(jnp.where(v == m, ii, IMAX))
        pred = lane == t
        rows = jnp.where(pred, am >> 11, rows)
        cols = jnp.where(pred, am & (N - 1), cols)
        v = jnp.where(ii == am, IMAX, v)
        return (v, rows, cols)

    zero = jnp.zeros((1, K), jnp.int32)
    _, rows, cols = lax.fori_loop(0, K, body, (v, zero, zero))
    row_ref[...] = rows
    col_ref[...] = cols


def _final_topk(cv, ci):
    return pl.pallas_call(
        _final_kernel,
        out_shape=(jax.ShapeDtypeStruct((1, K), jnp.int32),
                   jax.ShapeDtypeStruct((1, K), jnp.int32)),
    )(cv.reshape(CAND // K, K), ci.reshape(CAND // K, K))


def kernel(adj, W):
    keys, rmin = _masked_gradient_keys(adj, W)
    cv, ci = _extract_sc(keys, rmin.reshape(-1))
    rows, cols = _final_topk(cv, ci)
    actions = jnp.stack([rows[0], cols[0]], axis=-1)
    return (actions, jnp.zeros((1,), dtype=jnp.float32))


# speedup vs baseline: 33.9927x; 1.1023x over previous
import functools

import numpy as np

import jax
import jax.numpy as jnp
from jax import lax
from jax.experimental import pallas as pl
from jax.experimental.pallas import tpu as pltpu
from jax.experimental.pallas import tpu_sc as plsc

N = 2048
K = 128
BM = 1024
BN = 1024
BK = 512

NW = 32
ROWS_W = N // NW
CAP_T = 256
CAND = NW * CAP_T
IMAX = np.int32(0x7FFFFFFF)
IMIN = np.int32(-0x80000000)
KEY_INF = np.int32(0x7F800000)


def _grad_mask_kernel(w_ik, a_jk, a_ki, w_kj, out_ref, rmin_ref, acc_ref):
    p = pl.program_id(0)
    k = pl.program_id(1)
    k_last = pl.num_programs(1) - 1
    i = (p + 1) // 2
    j = p // 2

    @pl.when(k == 0)
    def _zero():
        acc_ref[...] = jnp.zeros_like(acc_ref)

    d1 = lax.dot_general(
        w_ik[...], a_jk[...], (((1,), (1,)), ((), ())),
        preferred_element_type=jnp.float32)
    d2 = lax.dot_general(
        a_ki[...], w_kj[...], (((0,), (0,)), ((), ())),
        preferred_element_type=jnp.float32)
    acc_ref[...] += d1 + d2

    @pl.when(k == k_last)
    def _mask():
        rows = i * BM + lax.broadcasted_iota(jnp.int32, (BM, BN), 0)
        cols = j * BN + lax.broadcasted_iota(jnp.int32, (BM, BN), 1)
        g = jnp.where(cols < rows, acc_ref[...], jnp.inf)
        u = lax.bitcast_convert_type(g, jnp.int32)
        skey = u ^ ((u >> 31) & IMAX)
        out_ref[...] = skey
        bmin = jnp.min(skey, axis=1, keepdims=True)

        @pl.when(j == 0)
        def _init():
            rmin_ref[pl.ds(i * BM, BM), :] = bmin

        @pl.when(j > 0)
        def _acc():
            rmin_ref[pl.ds(i * BM, BM), :] = jnp.minimum(
                rmin_ref[pl.ds(i * BM, BM), :], bmin)


def _masked_gradient_keys(adj, W):
    grid = (3, N // BK)
    return pl.pallas_call(
        _grad_mask_kernel,
        grid=grid,
        in_specs=[
            pl.BlockSpec((BM, BK), lambda p, k: ((p + 1) // 2, k)),
            pl.BlockSpec((BN, BK), lambda p, k: (p // 2, k)),
            pl.BlockSpec((BK, BM), lambda p, k: (k, (p + 1) // 2)),
            pl.BlockSpec((BK, BN), lambda p, k: (k, p // 2)),
        ],
        out_specs=(
            pl.BlockSpec((BM, BN), lambda p, k: ((p + 1) // 2, p // 2)),
            pl.BlockSpec((N, 1), lambda p, k: (0, 0)),
        ),
        out_shape=(jax.ShapeDtypeStruct((N, N), jnp.int32),
                   jax.ShapeDtypeStruct((N, 1), jnp.int32)),
        scratch_shapes=[pltpu.VMEM((BM, BN), jnp.float32)],
    )(W, adj, adj, W)


def _extract_sc(kflat, rmin):
    mesh = plsc.VectorSubcoreMesh(core_axis_name="c", subcore_axis_name="s")

    @functools.partial(
        pl.kernel,
        mesh=mesh,
        compiler_params=pltpu.CompilerParams(needs_layout_passes=False),
        out_type=(jax.ShapeDtypeStruct((CAND,), jnp.int32),
                  jax.ShapeDtypeStruct((CAND,), jnp.int32)),
        scratch_types=[
            pltpu.VMEM((N,), jnp.int32),
            pltpu.VMEM((N,), jnp.int32),
            pltpu.VMEM((CAP_T + 16,), jnp.int32),
            pltpu.VMEM((CAP_T + 16,), jnp.int32),
        ],
    )
    def ext_kernel(k_hbm, rm_hbm, out_v, out_i, rmv, rowbuf, cv, ci):
        wid = _worker_id()
        lanes = lax.iota(jnp.int32, 16)
        zero16 = jnp.zeros((16,), jnp.int32)
        imax16 = jnp.full((16,), IMAX, jnp.int32)

        pltpu.sync_copy(rm_hbm, rmv)

        def bs_body(_, carry):
            lo, hi = carry
            mid = lo + lax.shift_right_logical(hi - lo, 1)

            def cnt(b, acc):
                rv = rmv[pl.ds(b * 16, 16)]
                return acc + jnp.where(rv <= mid, 1, 0).astype(jnp.int32)
            acc = lax.fori_loop(0, N // 16, cnt, zero16, unroll=8)
            big = jnp.sum(acc) >= K
            return (jnp.where(big, lo, mid), jnp.where(big, mid, hi))
        _, thresh = lax.fori_loop(0, 32, bs_body, (IMIN, IMAX))

        def c_init(b, _):
            cv[pl.ds(b * 16, 16)] = imax16
            ci[pl.ds(b * 16, 16)] = zero16
            return 0
        lax.fori_loop(0, (CAP_T + 16) // 16, c_init, 0)

        def scan_row(row, ptr):
            pltpu.sync_copy(k_hbm.at[row], rowbuf)

            def v_body(i, ptr):
                sk = rowbuf[pl.ds(i * 16, 16)]
                colv = i * 16 + lanes
                m = (sk <= thresh) & (colv < row)
                idxv = row * N + colv
                p = jnp.minimum(ptr, CAP_T)
                plsc.store_compressed(cv.at[pl.ds(p, 16)], sk, mask=m)
                plsc.store_compressed(ci.at[pl.ds(p, 16)], idxv, mask=m)
                cnt16 = plsc.all_reduce_population_count(m)
                return ptr + cnt16[0]
            return lax.fori_loop(0, (row + 15) >> 4, v_body, ptr)

        def group_body(g, ptr):
            rv = rmv[pl.ds(wid * ROWS_W + g * 16, 16)]
            for l in range(16):
                row = wid * ROWS_W + g * 16 + l
                ptr = lax.cond(rv[l] <= thresh,
                               functools.partial(scan_row, row),
                               lambda p: p, ptr)
            return ptr
        lax.fori_loop(0, ROWS_W // 16, group_body, jnp.int32(0))

        pltpu.sync_copy(cv.at[pl.ds(0, CAP_T)],
                        out_v.at[pl.ds(wid * CAP_T, CAP_T)])
        pltpu.sync_copy(ci.at[pl.ds(0, CAP_T)],
                        out_i.at[pl.ds(wid * CAP_T, CAP_T)])

    return ext_kernel(kflat, rmin)


def _worker_id():
    return lax.axis_index("s") * 2 + lax.axis_index("c")


def _final_kernel(v_ref, i_ref, row_ref, col_ref):
    v = v_ref[...]
    ii = i_ref[...]
    lane = lax.broadcasted_iota(jnp.int32, (1, K), 1)

    def body(t, carry):
        v, rows, cols = carry
        m = jnp.min(v)
        am = jnp.min(jnp.where(v == m, ii, IMAX))
        pred = lane == t
        rows = jnp.where(pred, am >> 11, rows)
        cols = jnp.where(pred, am & (N - 1), cols)
        v = jnp.where(ii == am, IMAX, v)
        return (v, rows, cols)

    zero = jnp.zeros((1, K), jnp.int32)
    _, rows, cols = lax.fori_loop(0, K, body, (v, zero, zero))
    row_ref[...] = rows
    col_ref[...] = cols


def _final_topk(cv, ci):
    return pl.pallas_call(
        _final_kernel,
        out_shape=(jax.ShapeDtypeStruct((1, K), jnp.int32),
                   jax.ShapeDtypeStruct((1, K), jnp.int32)),
    )(cv.reshape(CAND // K, K), ci.reshape(CAND // K, K))


def kernel(adj, W):
    keys, rmin = _masked_gradient_keys(adj, W)
    cv, ci = _extract_sc(keys, rmin.reshape(-1))
    rows, cols = _final_topk(cv, ci)
    actions = jnp.stack([rows[0], cols[0]], axis=-1)
    return (actions, jnp.zeros((1,), dtype=jnp.float32))


# speedup vs baseline: 35.3137x; 1.0389x over previous
import functools

import numpy as np

import jax
import jax.numpy as jnp
from jax import lax
from jax.experimental import pallas as pl
from jax.experimental.pallas import tpu as pltpu
from jax.experimental.pallas import tpu_sc as plsc

N = 2048
K = 128
BM = 1024
BN = 1024
BK = 512

NW = 32
ROWS_W = N // NW
CAP_T = 256
CAND = NW * CAP_T
IMAX = np.int32(0x7FFFFFFF)
IMIN = np.int32(-0x80000000)
KEY_INF = np.int32(0x7F800000)


def _grad_mask_kernel(w_ref, a_ref, out_ref, rmin_ref):
    p = pl.program_id(0)
    i = (p + 1) // 2
    j = p // 2

    d1 = lax.dot_general(
        w_ref[pl.ds(i * BM, BM), :], a_ref[pl.ds(j * BN, BN), :],
        (((1,), (1,)), ((), ())), preferred_element_type=jnp.float32)
    d2 = lax.dot_general(
        a_ref[:, pl.ds(i * BM, BM)], w_ref[:, pl.ds(j * BN, BN)],
        (((0,), (0,)), ((), ())), preferred_element_type=jnp.float32)

    rows = i * BM + lax.broadcasted_iota(jnp.int32, (BM, BN), 0)
    cols = j * BN + lax.broadcasted_iota(jnp.int32, (BM, BN), 1)
    g = jnp.where(cols < rows, d1 + d2, jnp.inf)
    u = lax.bitcast_convert_type(g, jnp.int32)
    skey = u ^ ((u >> 31) & IMAX)
    out_ref[...] = skey
    bmin = jnp.min(skey, axis=1, keepdims=True)

    @pl.when(j == 0)
    def _init():
        rmin_ref[pl.ds(i * BM, BM), :] = bmin

    @pl.when(j > 0)
    def _acc():
        rmin_ref[pl.ds(i * BM, BM), :] = jnp.minimum(
            rmin_ref[pl.ds(i * BM, BM), :], bmin)


def _masked_gradient_keys(adj, W):
    return pl.pallas_call(
        _grad_mask_kernel,
        grid=(3,),
        in_specs=[
            pl.BlockSpec((N, N), lambda p: (0, 0)),
            pl.BlockSpec((N, N), lambda p: (0, 0)),
        ],
        out_specs=(
            pl.BlockSpec((BM, BN), lambda p: ((p + 1) // 2, p // 2)),
            pl.BlockSpec((N, 1), lambda p: (0, 0)),
        ),
        out_shape=(jax.ShapeDtypeStruct((N, N), jnp.int32),
                   jax.ShapeDtypeStruct((N, 1), jnp.int32)),
    )(W, adj)


def _extract_sc(kflat, rmin):
    mesh = plsc.VectorSubcoreMesh(core_axis_name="c", subcore_axis_name="s")

    @functools.partial(
        pl.kernel,
        mesh=mesh,
        compiler_params=pltpu.CompilerParams(needs_layout_passes=False),
        out_type=(jax.ShapeDtypeStruct((CAND,), jnp.int32),
                  jax.ShapeDtypeStruct((CAND,), jnp.int32)),
        scratch_types=[
            pltpu.VMEM((N,), jnp.int32),
            pltpu.VMEM((N,), jnp.int32),
            pltpu.VMEM((CAP_T + 16,), jnp.int32),
            pltpu.VMEM((CAP_T + 16,), jnp.int32),
        ],
    )
    def ext_kernel(k_hbm, rm_hbm, out_v, out_i, rmv, rowbuf, cv, ci):
        wid = _worker_id()
        lanes = lax.iota(jnp.int32, 16)
        zero16 = jnp.zeros((16,), jnp.int32)
        imax16 = jnp.full((16,), IMAX, jnp.int32)

        pltpu.sync_copy(rm_hbm, rmv)

        def bs_body(_, carry):
            lo, hi = carry
            mid = lo + lax.shift_right_logical(hi - lo, 1)

            def cnt(b, acc):
                rv = rmv[pl.ds(b * 16, 16)]
                return acc + jnp.where(rv <= mid, 1, 0).astype(jnp.int32)
            acc = lax.fori_loop(0, N // 16, cnt, zero16, unroll=8)
            big = jnp.sum(acc) >= K
            return (jnp.where(big, lo, mid), jnp.where(big, mid, hi))
        _, thresh = lax.fori_loop(0, 32, bs_body, (IMIN, IMAX))

        def c_init(b, _):
            cv[pl.ds(b * 16, 16)] = imax16
            ci[pl.ds(b * 16, 16)] = zero16
            return 0
        lax.fori_loop(0, (CAP_T + 16) // 16, c_init, 0)

        def scan_row(row, ptr):
            pltpu.sync_copy(k_hbm.at[row], rowbuf)

            def v_body(i, ptr):
                sk = rowbuf[pl.ds(i * 16, 16)]
                colv = i * 16 + lanes
                m = (sk <= thresh) & (colv < row)
                idxv = row * N + colv
                p = jnp.minimum(ptr, CAP_T)
                plsc.store_compressed(cv.at[pl.ds(p, 16)], sk, mask=m)
                plsc.store_compressed(ci.at[pl.ds(p, 16)], idxv, mask=m)
                cnt16 = plsc.all_reduce_population_count(m)
                return ptr + cnt16[0]
            return lax.fori_loop(0, (row + 15) >> 4, v_body, ptr)

        def group_body(g, ptr):
            rv = rmv[pl.ds(wid * ROWS_W + g * 16, 16)]
            for l in range(16):
                row = wid * ROWS_W + g * 16 + l
                ptr = lax.cond(rv[l] <= thresh,
                               functools.partial(scan_row, row),
                               lambda p: p, ptr)
            return ptr
        lax.fori_loop(0, ROWS_W // 16, group_body, jnp.int32(0))

        pltpu.sync_copy(cv.at[pl.ds(0, CAP_T)],
                        out_v.at[pl.ds(wid * CAP_T, CAP_T)])
        pltpu.sync_copy(ci.at[pl.ds(0, CAP_T)],
                        out_i.at[pl.ds(wid * CAP_T, CAP_T)])

    return ext_kernel(kflat, rmin)


def _worker_id():
    return lax.axis_index("s") * 2 + lax.axis_index("c")


def _final_kernel(v_ref, i_ref, row_ref, col_ref):
    v = v_ref[...]
    ii = i_ref[...]
    lane = lax.broadcasted_iota(jnp.int32, (1, K), 1)

    def body(t, carry):
        v, rows, cols = carry
        m = jnp.min(v)
        am = jnp.min(jnp.where(v == m, ii, IMAX))
        pred = lane == t
        rows = jnp.where(pred, am >> 11, rows)
        cols = jnp.where(pred, am & (N - 1), cols)
        v = jnp.where(ii == am, IMAX, v)
        return (v, rows, cols)

    zero = jnp.zeros((1, K), jnp.int32)
    _, rows, cols = lax.fori_loop(0, K, body, (v, zero, zero))
    row_ref[...] = rows
    col_ref[...] = cols


def _final_topk(cv, ci):
    return pl.pallas_call(
        _final_kernel,
        out_shape=(jax.ShapeDtypeStruct((1, K), jnp.int32),
                   jax.ShapeDtypeStruct((1, K), jnp.int32)),
    )(cv.reshape(CAND // K, K), ci.reshape(CAND // K, K))


def kernel(adj, W):
    keys, rmin = _masked_gradient_keys(adj, W)
    cv, ci = _extract_sc(keys, rmin.reshape(-1))
    rows, cols = _final_topk(cv, ci)
    actions = jnp.stack([rows[0], cols[0]], axis=-1)
    return (actions, jnp.zeros((1,), dtype=jnp.float32))


# speedup vs baseline: 52.0236x; 1.4732x over previous
import functools

import numpy as np

import jax
import jax.numpy as jnp
from jax import lax
from jax.experimental import pallas as pl
from jax.experimental.pallas import tpu as pltpu
from jax.experimental.pallas import tpu_sc as plsc

N = 2048
K = 128
BM = 1024
BN = 1024
BK = 512

NW = 32
ROWS_W = N // NW
CAP_T = 64
CAND = NW * CAP_T
IMAX = np.int32(0x7FFFFFFF)
IMIN = np.int32(-0x80000000)
KEY_INF = np.int32(0x7F800000)


def _grad_mask_kernel(w_ref, a_ref, out_ref, rmin_ref):
    p = pl.program_id(0)
    i = (p + 1) // 2
    j = p // 2

    d1 = lax.dot_general(
        w_ref[pl.ds(i * BM, BM), :], a_ref[pl.ds(j * BN, BN), :],
        (((1,), (1,)), ((), ())), preferred_element_type=jnp.float32)
    d2 = lax.dot_general(
        a_ref[:, pl.ds(i * BM, BM)], w_ref[:, pl.ds(j * BN, BN)],
        (((0,), (0,)), ((), ())), preferred_element_type=jnp.float32)

    rows = i * BM + lax.broadcasted_iota(jnp.int32, (BM, BN), 0)
    cols = j * BN + lax.broadcasted_iota(jnp.int32, (BM, BN), 1)
    g = jnp.where(cols < rows, d1 + d2, jnp.inf)
    u = lax.bitcast_convert_type(g, jnp.int32)
    skey = u ^ ((u >> 31) & IMAX)
    out_ref[...] = skey
    bmin = jnp.min(skey, axis=1, keepdims=True).T

    @pl.when(j == 0)
    def _init():
        rmin_ref[:, pl.ds(i * BM, BM)] = bmin

    @pl.when(j > 0)
    def _acc():
        rmin_ref[:, pl.ds(i * BM, BM)] = jnp.minimum(
            rmin_ref[:, pl.ds(i * BM, BM)], bmin)


def _masked_gradient_keys(adj, W):
    return pl.pallas_call(
        _grad_mask_kernel,
        grid=(3,),
        in_specs=[
            pl.BlockSpec((N, N), lambda p: (0, 0)),
            pl.BlockSpec((N, N), lambda p: (0, 0)),
        ],
        out_specs=(
            pl.BlockSpec((BM, BN), lambda p: ((p + 1) // 2, p // 2)),
            pl.BlockSpec((1, N), lambda p: (0, 0)),
        ),
        out_shape=(jax.ShapeDtypeStruct((N, N), jnp.int32),
                   jax.ShapeDtypeStruct((1, N), jnp.int32)),
    )(W, adj)


def _extract_sc(kflat, rmin):
    mesh = plsc.VectorSubcoreMesh(core_axis_name="c", subcore_axis_name="s")

    @functools.partial(
        pl.kernel,
        mesh=mesh,
        compiler_params=pltpu.CompilerParams(needs_layout_passes=False),
        out_type=(jax.ShapeDtypeStruct((1, CAND), jnp.int32),
                  jax.ShapeDtypeStruct((1, CAND), jnp.int32)),
        scratch_types=[
            pltpu.VMEM((N,), jnp.int32),
            pltpu.VMEM((N,), jnp.int32),
            pltpu.VMEM((CAP_T + 16,), jnp.int32),
            pltpu.VMEM((CAP_T + 16,), jnp.int32),
        ],
    )
    def ext_kernel(k_hbm, rm_hbm, out_v, out_i, rmv, rowbuf, cv, ci):
        wid = _worker_id()
        lanes = lax.iota(jnp.int32, 16)
        zero16 = jnp.zeros((16,), jnp.int32)
        imax16 = jnp.full((16,), IMAX, jnp.int32)

        pltpu.sync_copy(rm_hbm.at[0], rmv)

        def bs_body(_, carry):
            lo, hi = carry
            mid = lo + lax.shift_right_logical(hi - lo, 1)

            def cnt(b, acc):
                rv = rmv[pl.ds(b * 16, 16)]
                return acc + jnp.where(rv <= mid, 1, 0).astype(jnp.int32)
            acc = lax.fori_loop(0, N // 16, cnt, zero16, unroll=8)
            big = jnp.sum(acc) >= K
            return (jnp.where(big, lo, mid), jnp.where(big, mid, hi))
        _, thresh = lax.fori_loop(0, 32, bs_body, (IMIN, IMAX))

        def c_init(b, _):
            cv[pl.ds(b * 16, 16)] = imax16
            ci[pl.ds(b * 16, 16)] = zero16
            return 0
        lax.fori_loop(0, (CAP_T + 16) // 16, c_init, 0)

        def scan_row(row, ptr):
            pltpu.sync_copy(k_hbm.at[row], rowbuf)

            def v_body(i, ptr):
                sk = rowbuf[pl.ds(i * 16, 16)]
                colv = i * 16 + lanes
                m = (sk <= thresh) & (colv < row)
                idxv = row * N + colv
                p = jnp.minimum(ptr, CAP_T)
                plsc.store_compressed(cv.at[pl.ds(p, 16)], sk, mask=m)
                plsc.store_compressed(ci.at[pl.ds(p, 16)], idxv, mask=m)
                cnt16 = plsc.all_reduce_population_count(m)
                return ptr + cnt16[0]
            return lax.fori_loop(0, (row + 15) >> 4, v_body, ptr)

        def group_body(g, ptr):
            rv = rmv[pl.ds(wid * ROWS_W + g * 16, 16)]
            for l in range(16):
                row = wid * ROWS_W + g * 16 + l
                ptr = lax.cond(rv[l] <= thresh,
                               functools.partial(scan_row, row),
                               lambda p: p, ptr)
            return ptr
        lax.fori_loop(0, ROWS_W // 16, group_body, jnp.int32(0))

        pltpu.sync_copy(cv.at[pl.ds(0, CAP_T)],
                        out_v.at[0, pl.ds(wid * CAP_T, CAP_T)])
        pltpu.sync_copy(ci.at[pl.ds(0, CAP_T)],
                        out_i.at[0, pl.ds(wid * CAP_T, CAP_T)])

    return ext_kernel(kflat, rmin)


def _worker_id():
    return lax.axis_index("s") * 2 + lax.axis_index("c")


def _final_kernel(v_ref, i_ref, out_ref):
    v = v_ref[...]
    ii = i_ref[...]
    a = v.reshape(CAND, 1)
    ai = ii.reshape(CAND, 1)

    rank = jnp.zeros((CAND, 1), jnp.int32)
    for r in range(CAND // 128):
        b = v[:, r * 128:(r + 1) * 128]
        bi = ii[:, r * 128:(r + 1) * 128]
        less = (b < a) | ((b == a) & (bi < ai))
        rank = rank + jnp.sum(less.astype(jnp.int32), axis=1, keepdims=True)

    t_iota = lax.broadcasted_iota(jnp.int32, (CAND, K), 1)
    onehot = (rank == t_iota).astype(jnp.float32)
    rowf = (ai >> 11).astype(jnp.float32)
    colf = (ai & (N - 1)).astype(jnp.float32)
    rc = jnp.concatenate([rowf, colf], axis=1)
    res = lax.dot_general(onehot, rc, (((0,), (0,)), ((), ())),
                          preferred_element_type=jnp.float32)
    out_ref[...] = res.astype(jnp.int32)


def _final_topk(cv, ci):
    return pl.pallas_call(
        _final_kernel,
        out_shape=jax.ShapeDtypeStruct((K, 2), jnp.int32),
    )(cv, ci)


def kernel(adj, W):
    keys, rmin = _masked_gradient_keys(adj, W)
    cv, ci = _extract_sc(keys, rmin)
    actions = _final_topk(cv, ci)
    return (actions, jnp.zeros((1,), dtype=jnp.float32))
